# Initial kernel scaffold; baseline (speedup 1.0000x reference)
import jax
import jax.numpy as jnp
from jax.experimental import pallas as pl


def kernel(learn, fix):
    raise NotImplementedError("write your pallas kernel here")


# speedup vs baseline: 1.2394x; 1.2394x over previous
import functools

import jax
import jax.numpy as jnp
from jax.experimental import pallas as pl
from jax.experimental.pallas import tpu as pltpu


def _loss_tile_kernel(learn_tile_ref, fix_tile_ref, fix_all_ref,
                      row_out_ref, col_out_ref, colsum_ref,
                      *, num_tiles, half_weight):
    i = pl.program_id(0)

    learn_t = learn_tile_ref[...]
    fix_t = fix_tile_ref[...]
    fix_all = fix_all_ref[...]

    dn = (((1,), (1,)), ((), ()))
    sim = jax.lax.dot_general(learn_t, fix_all, dn,
                              preferred_element_type=jnp.float32)
    e = jnp.exp(sim)

    row_sum = jnp.sum(e, axis=1, keepdims=True)
    diag = jnp.sum(learn_t.astype(jnp.float32) * fix_t.astype(jnp.float32),
                   axis=1, keepdims=True)
    row_out_ref[...] = half_weight * jnp.log(row_sum) - (2.0 * half_weight) * diag

    @pl.when(i == 0)
    def _init():
        colsum_ref[...] = jnp.zeros_like(colsum_ref)
    colsum_ref[...] += jnp.sum(e, axis=0, keepdims=True)

    @pl.when(i == num_tiles - 1)
    def _finish():
        col_out_ref[...] = half_weight * jnp.log(colsum_ref[...])


def _pick_row_tile(n):
    for t in (512, 256, 128, 64, 32, 16, 8):
        if n % t == 0:
            return t
    return n


def _t2t_loss(learn, fix, *, temperature=0.07, loss_weight=1.0, tm=None):
    assert learn.ndim == 2 and learn.shape == fix.shape
    n, d = learn.shape
    inv_temp = 1.0 / temperature

    learn_s = (learn.astype(jnp.float32) * inv_temp).astype(jnp.bfloat16)
    fix_b = fix.astype(jnp.bfloat16)

    if tm is None:
        tm = _pick_row_tile(n)
    num_tiles = n // tm

    body = functools.partial(
        _loss_tile_kernel,
        num_tiles=num_tiles,
        half_weight=0.5 * float(loss_weight),
    )

    row_out, col_out = pl.pallas_call(
        body,
        out_shape=(jax.ShapeDtypeStruct((n, 1), jnp.float32),
                   jax.ShapeDtypeStruct((1, n), jnp.float32)),
        grid=(num_tiles,),
        in_specs=[
            pl.BlockSpec((tm, d), lambda i: (i, 0)),
            pl.BlockSpec((tm, d), lambda i: (i, 0)),
            pl.BlockSpec((n, d), lambda i: (0, 0)),
        ],
        out_specs=(
            pl.BlockSpec((tm, 1), lambda i: (i, 0)),
            pl.BlockSpec((1, n), lambda i: (0, 0)),
        ),
        scratch_shapes=[pltpu.VMEM((1, n), jnp.float32)],
        compiler_params=pltpu.CompilerParams(
            dimension_semantics=("arbitrary",),
            vmem_limit_bytes=64 * 2 ** 20),
    )(learn_s, fix_b, fix_b)

    per_class = row_out[:, 0] + col_out[0, :]
    return jnp.mean(per_class)


def kernel(learn, fix):
    return _t2t_loss(learn, fix, temperature=0.07, loss_weight=1.0)


# speedup vs baseline: 2.0901x; 1.6864x over previous
import functools

import jax
import jax.numpy as jnp
from jax.experimental import pallas as pl
from jax.experimental.pallas import tpu as pltpu


def _loss_kernel(learn_tile_ref, fix_all_ref, out_ref,
                 fix_bf_ref, colsum_ref, rowacc_ref,
                 *, num_tiles, tm, inv_temp, half_weight):
    i = pl.program_id(0)

    @pl.when(i == 0)
    def _init():
        fix_bf_ref[...] = fix_all_ref[...].astype(jnp.bfloat16)
        colsum_ref[...] = jnp.zeros_like(colsum_ref)
        rowacc_ref[0, 0] = 0.0

    learn_t32 = learn_tile_ref[...] * inv_temp
    learn_bf = learn_t32.astype(jnp.bfloat16)

    dn = (((1,), (1,)), ((), ()))
    sim = jax.lax.dot_general(learn_bf, fix_bf_ref[...], dn,
                              preferred_element_type=jnp.float32)
    e = jnp.exp(sim)

    row_sum = jnp.sum(e, axis=1, keepdims=True)
    fix_t32 = fix_all_ref[pl.ds(i * tm, tm), :]
    diag = jnp.sum(learn_t32 * fix_t32, axis=1, keepdims=True)
    rowacc_ref[0, 0] += jnp.sum(half_weight * jnp.log(row_sum) - diag)

    colsum_ref[...] += jnp.sum(e, axis=0, keepdims=True)

    @pl.when(i == num_tiles - 1)
    def _finish():
        col_total = jnp.sum(half_weight * jnp.log(colsum_ref[...]))
        out_ref[0, 0] = (rowacc_ref[0, 0] + col_total) / (num_tiles * tm)


def _pick_row_tile(n):
    for t in (512, 256, 128, 64, 32, 16, 8):
        if n % t == 0:
            return t
    return n


def _t2t_loss(learn, fix, *, temperature=0.07, loss_weight=1.0, tm=None):
    assert learn.ndim == 2 and learn.shape == fix.shape
    n, d = learn.shape

    if tm is None:
        tm = _pick_row_tile(n)
    num_tiles = n // tm

    body = functools.partial(
        _loss_kernel,
        num_tiles=num_tiles,
        tm=tm,
        inv_temp=1.0 / temperature,
        half_weight=0.5 * float(loss_weight),
    )

    out = pl.pallas_call(
        body,
        out_shape=jax.ShapeDtypeStruct((1, 1), jnp.float32),
        grid=(num_tiles,),
        in_specs=[
            pl.BlockSpec((tm, d), lambda i: (i, 0)),
            pl.BlockSpec((n, d), lambda i: (0, 0)),
        ],
        out_specs=pl.BlockSpec(memory_space=pltpu.SMEM),
        scratch_shapes=[
            pltpu.VMEM((n, d), jnp.bfloat16),
            pltpu.VMEM((1, n), jnp.float32),
            pltpu.SMEM((1, 1), jnp.float32),
        ],
        compiler_params=pltpu.CompilerParams(
            dimension_semantics=("arbitrary",),
            vmem_limit_bytes=64 * 2 ** 20),
    )(learn.astype(jnp.float32), fix.astype(jnp.float32))

    return jnp.reshape(out, ())


def kernel(learn, fix):
    return _t2t_loss(learn, fix, temperature=0.07, loss_weight=1.0)


# speedup vs baseline: 2.0920x; 1.0009x over previous
import functools

import jax
import jax.numpy as jnp
from jax.experimental import pallas as pl
from jax.experimental.pallas import tpu as pltpu


def _loss_kernel(learn_tile_ref, fix_all_ref, out_ref,
                 fix_bf_ref, colsum_ref, rowacc_ref,
                 *, num_tiles, tm, cn, inv_temp, half_weight):
    i = pl.program_id(0)
    n = fix_all_ref.shape[0]

    @pl.when(i == 0)
    def _init():
        fix_bf_ref[...] = fix_all_ref[...].astype(jnp.bfloat16)
        colsum_ref[...] = jnp.zeros_like(colsum_ref)
        rowacc_ref[0, 0] = 0.0

    learn_t32 = learn_tile_ref[...] * inv_temp
    learn_bf = learn_t32.astype(jnp.bfloat16)

    dn = (((1,), (1,)), ((), ()))

    row_sum = jnp.zeros((tm, 1), jnp.float32)
    for c in range(n // cn):
        fb = fix_bf_ref[pl.ds(c * cn, cn), :]
        sim_c = jax.lax.dot_general(learn_bf, fb, dn,
                                    preferred_element_type=jnp.float32)
        e_c = jnp.exp(sim_c)
        row_sum = row_sum + jnp.sum(e_c, axis=1, keepdims=True)
        colsum_ref[:, pl.ds(c * cn, cn)] += jnp.sum(e_c, axis=0, keepdims=True)

    fix_t32 = fix_all_ref[pl.ds(i * tm, tm), :]
    diag = jnp.sum(learn_t32 * fix_t32, axis=1, keepdims=True)
    rowacc_ref[0, 0] += jnp.sum(half_weight * jnp.log(row_sum) - diag)

    @pl.when(i == num_tiles - 1)
    def _finish():
        col_total = jnp.sum(half_weight * jnp.log(colsum_ref[...]))
        out_ref[0, 0] = (rowacc_ref[0, 0] + col_total) / (num_tiles * tm)


def _pick_row_tile(n):
    for t in (512, 256, 128, 64, 32, 16, 8):
        if n % t == 0:
            return t
    return n


def _t2t_loss(learn, fix, *, temperature=0.07, loss_weight=1.0, tm=None,
              cn=None):
    assert learn.ndim == 2 and learn.shape == fix.shape
    n, d = learn.shape

    if tm is None:
        tm = _pick_row_tile(n)
    if cn is None:
        cn = min(n, 2048)
    if n % cn != 0:
        cn = n
    num_tiles = n // tm

    body = functools.partial(
        _loss_kernel,
        num_tiles=num_tiles,
        tm=tm,
        cn=cn,
        inv_temp=1.0 / temperature,
        half_weight=0.5 * float(loss_weight),
    )

    out = pl.pallas_call(
        body,
        out_shape=jax.ShapeDtypeStruct((1, 1), jnp.float32),
        grid=(num_tiles,),
        in_specs=[
            pl.BlockSpec((tm, d), lambda i: (i, 0)),
            pl.BlockSpec((n, d), lambda i: (0, 0)),
        ],
        out_specs=pl.BlockSpec(memory_space=pltpu.SMEM),
        scratch_shapes=[
            pltpu.VMEM((n, d), jnp.bfloat16),
            pltpu.VMEM((1, n), jnp.float32),
            pltpu.SMEM((1, 1), jnp.float32),
        ],
        compiler_params=pltpu.CompilerParams(
            dimension_semantics=("arbitrary",),
            vmem_limit_bytes=64 * 2 ** 20),
    )(learn.astype(jnp.float32), fix.astype(jnp.float32))

    return jnp.reshape(out, ())


def kernel(learn, fix):
    return _t2t_loss(learn, fix, temperature=0.07, loss_weight=1.0)
